# baseline (speedup 1.0000x reference)
import jax
import jax.numpy as jnp
from jax import lax
from jax.experimental import pallas as pl
from jax.experimental.pallas import tpu as pltpu
from jax.experimental.pallas import tpu_sc as plsc

_D = 64
_W = 128
_L = 16
_EPS = 1e-5
_NC = 2
_NS = 16
_NW = _NC * _NS
_CHUNK = 128
_NG = 4
_NS_BUF = 2

_GDN = lax.GatherDimensionNumbers(
    offset_dims=(), collapsed_slice_dims=(0,), start_index_map=(0,))


def _shuffle(v, p2d):
    return lax.gather(v, p2d, _GDN, slice_sizes=(1,),
                      mode=lax.GatherScatterMode.PROMISE_IN_BOUNDS)


def _ln_body(x_hbm, tpad_hbm, gamma_hbm, beta_hbm, out_hbm,
             idx_all, gbuf, sbuf, gsem, ssem, gam_v, bet_v):
    total = x_hbm.shape[0]
    rpw = total // _NW
    nchunk = rpw // _CHUNK
    wid = lax.axis_index("s") * _NC + lax.axis_index("c")
    base = wid * rpw

    pltpu.sync_copy(x_hbm.at[pl.ds(base, rpw)], idx_all)
    pltpu.sync_copy(gamma_hbm, gam_v)
    pltpu.sync_copy(beta_hbm, bet_v)

    iota = lax.iota(jnp.int32, _L)
    perms = [(iota ^ jnp.int32(1 << k)).reshape(_L, 1) for k in range(4)]
    gs = [gam_v[pl.ds(d * _L, _L)] for d in range(4)]
    bs = [bet_v[pl.ds(d * _L, _L)] for d in range(4)]
    ones_i = jnp.full((_L,), 1, jnp.int32)

    def start_gather(g, b):
        pltpu.async_copy(
            tpad_hbm.at[idx_all.at[pl.ds(g * _CHUNK, _CHUNK)]],
            gbuf[b], gsem[b])

    def wait_gather(b):
        pltpu.make_async_copy(
            tpad_hbm.at[pl.ds(0, _CHUNK)], gbuf[b], gsem[b]).wait()

    def start_store(g, s):
        pltpu.async_copy(
            sbuf[s], out_hbm.at[pl.ds(base + g * _CHUNK, _CHUNK)], ssem[s])

    def wait_store(s):
        pltpu.make_async_copy(
            sbuf[s], out_hbm.at[pl.ds(0, _CHUNK)], ssem[s]).wait()

    def compute(b, s):
        gb = gbuf[b]
        sb = sbuf[s]

        @plsc.parallel_loop(0, _CHUNK, unroll=8)
        def _row(r):
            vs = [gb[r, pl.ds(d * _L, _L)] for d in range(4)]
            ssum = (vs[0] + vs[1]) + (vs[2] + vs[3])
            q = (vs[0] * vs[0] + vs[1] * vs[1]) + (
                vs[2] * vs[2] + vs[3] * vs[3])
            for p in perms:
                ssum = ssum + _shuffle(ssum, p)
                q = q + _shuffle(q, p)
            mean = ssum * jnp.float32(1.0 / _D)
            var = q * jnp.float32(1.0 / _D) - mean * mean + jnp.float32(_EPS)
            ibits = lax.bitcast_convert_type(var, jnp.int32)
            ibits = jnp.int32(0x5F3759DF) - lax.shift_right_arithmetic(
                ibits, ones_i)
            y = lax.bitcast_convert_type(ibits, jnp.float32)
            half = var * jnp.float32(0.5)
            y = y * (jnp.float32(1.5) - half * y * y)
            y = y * (jnp.float32(1.5) - half * y * y)
            for d in range(4):
                sb[r, pl.ds(d * _L, _L)] = (vs[d] - mean) * y * gs[d] + bs[d]

    for g in range(_NG):
        start_gather(g, g)

    @pl.loop(0, nchunk, step=_NG)
    def _blk(g0):
        for db in range(_NG):
            g = g0 + db
            s = db % _NS_BUF
            wait_gather(db)

            @pl.when(g >= _NS_BUF)
            def _ws():
                wait_store(s)

            compute(db, s)
            start_store(g, s)

            @pl.when(g + _NG < nchunk)
            def _pf():
                start_gather(g + _NG, db)

    for s in range(_NS_BUF):
        wait_store(s)


def kernel(x, table, gamma, beta):
    b, h = x.shape
    total = b * h
    xf = x.reshape(total)
    tpad = jnp.pad(table, ((0, 0), (0, _W - _D)))
    rpw = total // _NW
    mesh = plsc.VectorSubcoreMesh(core_axis_name="c", subcore_axis_name="s")
    fn = pl.kernel(
        _ln_body,
        out_type=jax.ShapeDtypeStruct((total, _D), jnp.float32),
        mesh=mesh,
        scratch_types=[
            pltpu.VMEM((rpw,), jnp.int32),
            [pltpu.VMEM((_CHUNK, _W), jnp.float32) for _ in range(_NG)],
            [pltpu.VMEM((_CHUNK, _D), jnp.float32) for _ in range(_NS_BUF)],
            [pltpu.SemaphoreType.DMA for _ in range(_NG)],
            [pltpu.SemaphoreType.DMA for _ in range(_NS_BUF)],
            pltpu.VMEM((_D,), jnp.float32),
            pltpu.VMEM((_D,), jnp.float32),
        ],
    )
    out = fn(xf, tpad, gamma, beta)
    return out.reshape(b, h, _D)


# speedup vs baseline: 1.0755x; 1.0755x over previous
import jax
import jax.numpy as jnp
from jax import lax
from jax.experimental import pallas as pl
from jax.experimental.pallas import tpu as pltpu
from jax.experimental.pallas import tpu_sc as plsc

_D = 64
_W = 128
_L = 16
_EPS = 1e-5
_NC = 2
_NS = 16
_NW = _NC * _NS
_CHUNK = 128
_NG = 4
_NS_BUF = 2

_GDN = lax.GatherDimensionNumbers(
    offset_dims=(), collapsed_slice_dims=(0,), start_index_map=(0,))


def _shuffle(v, p2d):
    return lax.gather(v, p2d, _GDN, slice_sizes=(1,),
                      mode=lax.GatherScatterMode.PROMISE_IN_BOUNDS)


def _ln_body(x_hbm, tpad_hbm, gamma_hbm, beta_hbm, out_hbm,
             idx_all, gbuf, sbuf, gsem, ssem, gam_v, bet_v):
    total = x_hbm.shape[0]
    rpw = total // _NW
    nchunk = rpw // _CHUNK
    wid = lax.axis_index("s") * _NC + lax.axis_index("c")
    base = wid * rpw

    pltpu.sync_copy(x_hbm.at[pl.ds(base, rpw)], idx_all)
    pltpu.sync_copy(gamma_hbm, gam_v)
    pltpu.sync_copy(beta_hbm, bet_v)

    iota = lax.iota(jnp.int32, _L)
    perms = [(iota ^ jnp.int32(1 << k)).reshape(_L, 1) for k in range(4)]
    gs = [gam_v[pl.ds(d * _L, _L)] for d in range(4)]
    bs = [bet_v[pl.ds(d * _L, _L)] for d in range(4)]
    ones_i = jnp.full((_L,), 1, jnp.int32)

    def start_gather(g, b):
        pltpu.async_copy(
            tpad_hbm.at[idx_all.at[pl.ds(g * _CHUNK, _CHUNK)]],
            gbuf[b], gsem[b])

    def wait_gather(b):
        pltpu.make_async_copy(
            tpad_hbm.at[pl.ds(0, _CHUNK)], gbuf[b], gsem[b]).wait()

    def start_store(g, s):
        pltpu.async_copy(
            sbuf[s], out_hbm.at[pl.ds(base + g * _CHUNK, _CHUNK)], ssem[s])

    def wait_store(s):
        pltpu.make_async_copy(
            sbuf[s], out_hbm.at[pl.ds(0, _CHUNK)], ssem[s]).wait()

    def compute(b, s):
        gb = gbuf[b]
        sb = sbuf[s]

        @plsc.parallel_loop(0, _CHUNK, unroll=4)
        def _row(r):
            vs = [gb[r, pl.ds(d * _L, _L)] for d in range(4)]
            ssum = (vs[0] + vs[1]) + (vs[2] + vs[3])
            q = (vs[0] * vs[0] + vs[1] * vs[1]) + (
                vs[2] * vs[2] + vs[3] * vs[3])
            for p in perms:
                ssum = ssum + _shuffle(ssum, p)
                q = q + _shuffle(q, p)
            mean = ssum * jnp.float32(1.0 / _D)
            var = q * jnp.float32(1.0 / _D) - mean * mean + jnp.float32(_EPS)
            ibits = lax.bitcast_convert_type(var, jnp.int32)
            ibits = jnp.int32(0x5F3759DF) - lax.shift_right_arithmetic(
                ibits, ones_i)
            y = lax.bitcast_convert_type(ibits, jnp.float32)
            half = var * jnp.float32(0.5)
            y = y * (jnp.float32(1.5) - half * y * y)
            y = y * (jnp.float32(1.5) - half * y * y)
            for d in range(4):
                sb[r, pl.ds(d * _L, _L)] = (vs[d] - mean) * y * gs[d] + bs[d]

    for g in range(_NG):
        start_gather(g, g)

    @pl.loop(0, nchunk, step=_NG)
    def _blk(g0):
        for db in range(_NG):
            g = g0 + db
            s = db % _NS_BUF
            wait_gather(db)

            @pl.when(g >= _NS_BUF)
            def _ws():
                wait_store(s)

            compute(db, s)
            start_store(g, s)

            @pl.when(g + _NG < nchunk)
            def _pf():
                start_gather(g + _NG, db)

    for s in range(_NS_BUF):
        wait_store(s)


def kernel(x, table, gamma, beta):
    b, h = x.shape
    total = b * h
    xf = x.reshape(total)
    tpad = jnp.pad(table, ((0, 0), (0, _W - _D)))
    rpw = total // _NW
    mesh = plsc.VectorSubcoreMesh(core_axis_name="c", subcore_axis_name="s")
    fn = pl.kernel(
        _ln_body,
        out_type=jax.ShapeDtypeStruct((total, _D), jnp.float32),
        mesh=mesh,
        scratch_types=[
            pltpu.VMEM((rpw,), jnp.int32),
            [pltpu.VMEM((_CHUNK, _W), jnp.float32) for _ in range(_NG)],
            [pltpu.VMEM((_CHUNK, _D), jnp.float32) for _ in range(_NS_BUF)],
            [pltpu.SemaphoreType.DMA for _ in range(_NG)],
            [pltpu.SemaphoreType.DMA for _ in range(_NS_BUF)],
            pltpu.VMEM((_D,), jnp.float32),
            pltpu.VMEM((_D,), jnp.float32),
        ],
    )
    out = fn(xf, tpad, gamma, beta)
    return out.reshape(b, h, _D)
